# baseline (speedup 1.0000x reference)
import functools

import jax
import jax.numpy as jnp
from jax import lax
from jax.experimental import pallas as pl
from jax.experimental.pallas import tpu as pltpu
from jax.experimental.pallas import tpu_sc as plsc

N = 10000
E = 160000
B = 4
NUM_UNITS = 64

_NC = 2
_NS = 16
_EK = 160
_EPT = 10240
_EP = _NS * _EPT
_NB = _EPT // _EK
_NP = 10240
_RPT = _NP // _NS
_FC = 128


@functools.cache
def _make_spmm(ntbl, plan):
    njob = len(plan)
    mesh = plsc.VectorSubcoreMesh(core_axis_name="c", subcore_axis_name="s")

    scratch = [
        pltpu.VMEM_SHARED((_NP, _FC), jnp.float32),
        pltpu.VMEM((_EK, _FC), jnp.float32),
        pltpu.VMEM((_EK,), jnp.int32),
        pltpu.VMEM((_EK,), jnp.int32),
        pltpu.VMEM((_EK + 16,), jnp.float32),
        pltpu.SemaphoreType.DMA,
    ]

    @functools.partial(
        pl.kernel,
        out_type=tuple(jax.ShapeDtypeStruct((_NP, _FC), jnp.float32)
                       for _ in range(njob)),
        mesh=mesh,
        scratch_types=scratch,
    )
    def spmm(*refs):
        tbls = refs[:ntbl]
        src_h, dst_h, w1_h, w2_h = refs[ntbl:ntbl + 4]
        outs = refs[ntbl + 4:ntbl + 4 + njob]
        acc, gbuf, gi, si, wv, semG = refs[ntbl + 4 + njob:]
        cid = lax.axis_index("c")
        sid = lax.axis_index("s")
        zv = jnp.zeros((16,), jnp.float32)

        for j in range(njob):
            ti, sup = plan[j]
            tbl, out = tbls[ti], outs[j]
            gidx_h = src_h if sup == 0 else dst_h
            sidx_h = dst_h if sup == 0 else src_h
            w_h = w1_h if sup == 0 else w2_h

            @pl.when(cid == (j % _NC))
            def _(tbl=tbl, out=out, gidx_h=gidx_h, sidx_h=sidx_h, w_h=w_h):
                row0 = sid * _RPT
                ebase = sid * _EPT

                def zrow(i, _):
                    for c in range(_FC // 16):
                        gbuf[i, pl.ds(c * 16, 16)] = zv
                    return 0

                lax.fori_loop(0, _EK, zrow, 0)
                for z in range(_RPT // _EK):
                    pltpu.sync_copy(gbuf,
                                    acc.at[pl.ds(row0 + z * _EK, _EK)])
                plsc.subcore_barrier()

                def blk(b, _):
                    base = ebase + b * _EK
                    pltpu.sync_copy(gidx_h.at[pl.ds(base, _EK)], gi)
                    pltpu.sync_copy(sidx_h.at[pl.ds(base, _EK)], si)
                    pltpu.sync_copy(w_h.at[pl.ds(base, _EK)],
                                    wv.at[pl.ds(0, _EK)])
                    pltpu.async_copy(tbl.at[gi], gbuf, semG).wait()

                    @plsc.parallel_loop(0, _EK, 1, unroll=4)
                    def erow(i):
                        ws = wv[pl.ds(i, 16)][0]
                        for c in range(_FC // 16):
                            sl = pl.ds(c * 16, 16)
                            gbuf[i, sl] = gbuf[i, sl] * ws

                    pltpu.sync_copy(gbuf, acc.at[si], add=True)
                    return 0

                lax.fori_loop(0, _NB, blk, 0)
                plsc.subcore_barrier()
                pltpu.sync_copy(acc.at[pl.ds(row0, _RPT)],
                                out.at[pl.ds(row0, _RPT)])
                plsc.subcore_barrier()

    return spmm


def _diffuse(tables, pk):
    nt = len(tables)
    plan = tuple((c, s) for c in range(nt) for s in (0, 1))
    fn = _make_spmm(nt, plan)
    outs = fn(*tables, *pk)
    outs = [o[:N] for o in outs]
    ys0 = [outs[2 * c] for c in range(nt)]
    ys1 = [outs[2 * c + 1] for c in range(nt)]
    return ys0, ys1


def _diffuse_pair(tablesA, tablesB, pk):
    nt = len(tablesA)
    assert len(tablesB) == nt
    plan = tuple((c * 2 + s, s) for c in range(nt) for s in (0, 1))
    fn = _make_spmm(2 * nt, plan)
    tbls = []
    for c in range(nt):
        tbls.append(tablesA[c])
        tbls.append(tablesB[c])
    outs = fn(*tbls, *pk)
    outs = [o[:N] for o in outs]
    ysA = [outs[2 * c] for c in range(nt)]
    ysB = [outs[2 * c + 1] for c in range(nt)]
    return ysA, ysB


_M = N * B
_RBLK = 2000


@functools.cache
def _make_gate(nt):
    def body(*refs):
        ts = refs[:nt]
        ws = refs[nt:2 * nt]
        bias = refs[2 * nt]
        h0, h1 = refs[2 * nt + 1], refs[2 * nt + 2]
        rh0, rh1, u = refs[2 * nt + 3:]
        acc = bias[...].astype(jnp.float32) * jnp.ones((_RBLK, 1), jnp.float32)
        for i in range(nt):
            acc += jnp.dot(ts[i][...], ws[i][...],
                           preferred_element_type=jnp.float32)
        g = jax.nn.sigmoid(acc)
        rh0[...] = g[:, :32] * h0[...]
        rh1[...] = g[:, 32:64] * h1[...]
        u[...] = g[:, 64:]

    t_spec = pl.BlockSpec((_RBLK, 32), lambda i: (i, 0))
    w_spec = pl.BlockSpec((32, 128), lambda i: (0, 0))

    return pl.pallas_call(
        body,
        grid=(_M // _RBLK,),
        in_specs=[t_spec] * nt + [w_spec] * nt
        + [pl.BlockSpec((1, 128), lambda i: (0, 0))]
        + [t_spec, t_spec],
        out_specs=[t_spec, t_spec, pl.BlockSpec((_RBLK, 64), lambda i: (i, 0))],
        out_shape=[
            jax.ShapeDtypeStruct((_M, 32), jnp.float32),
            jax.ShapeDtypeStruct((_M, 32), jnp.float32),
            jax.ShapeDtypeStruct((_M, 64), jnp.float32),
        ],
    )


@functools.cache
def _make_cand(nt):
    def body(*refs):
        ts = refs[:nt]
        ws = refs[nt:2 * nt]
        bias = refs[2 * nt]
        u = refs[2 * nt + 1]
        h0, h1 = refs[2 * nt + 2], refs[2 * nt + 3]
        hp0, hp1 = refs[2 * nt + 4:]
        acc = bias[...].astype(jnp.float32) * jnp.ones((_RBLK, 1), jnp.float32)
        for i in range(nt):
            acc += jnp.dot(ts[i][...], ws[i][...],
                           preferred_element_type=jnp.float32)
        c = jnp.tanh(acc)
        uu = u[...]
        hp0[...] = uu[:, :32] * h0[...] + (1.0 - uu[:, :32]) * c[:, :32]
        hp1[...] = uu[:, 32:] * h1[...] + (1.0 - uu[:, 32:]) * c[:, 32:]

    t_spec = pl.BlockSpec((_RBLK, 32), lambda i: (i, 0))
    w_spec = pl.BlockSpec((32, 64), lambda i: (0, 0))

    return pl.pallas_call(
        body,
        grid=(_M // _RBLK,),
        in_specs=[t_spec] * nt + [w_spec] * nt
        + [pl.BlockSpec((1, 64), lambda i: (0, 0))]
        + [pl.BlockSpec((_RBLK, 64), lambda i: (i, 0)), t_spec, t_spec],
        out_specs=[t_spec, t_spec],
        out_shape=[
            jax.ShapeDtypeStruct((_M, 32), jnp.float32),
            jax.ShapeDtypeStruct((_M, 32), jnp.float32),
        ],
    )


def _adjust_weights(W, chans):
    C = W.shape[0] // 5
    out = W.shape[1]
    Wr = W.reshape(C, 5, out)
    Wa = jnp.stack([
        Wr[:, 0] - Wr[:, 2] - Wr[:, 4],
        Wr[:, 1],
        2.0 * Wr[:, 2],
        Wr[:, 3],
        2.0 * Wr[:, 4],
    ])
    res = []
    for idx in chans:
        ia = jnp.array([max(ch, 0) for ch in idx], jnp.int32)
        mask = jnp.array([1.0 if ch >= 0 else 0.0 for ch in idx], W.dtype)
        res.append(jnp.take(Wa, ia, axis=1) * mask[None, :, None])
    return res


def _flat(t):
    return t.reshape(_M, 32)


def kernel(inputs, hidden_state, src, dst, w1, w2, Wg0, bg0, Wc0, bc0, Wg1, bg1, Wc1, bc1):
    f32 = jnp.float32
    pe = _EP - E
    pk = (jnp.pad(src.astype(jnp.int32), (0, pe)),
          jnp.pad(dst.astype(jnp.int32), (0, pe)),
          jnp.pad(w1, (0, pe)), jnp.pad(w2, (0, pe)))
    x_nb = jnp.transpose(inputs, (1, 0, 2))
    x_tbl = jnp.pad(x_nb, ((0, 0), (0, 0), (0, 30))).reshape(N, 128)

    def h_chunks(h):
        t = jnp.transpose(h, (1, 0, 2)).reshape(N, B, 2, 32)
        t = jnp.transpose(t, (0, 2, 1, 3))
        return [t[:, 0].reshape(N, 128), t[:, 1].reshape(N, 128)]

    h0c = h_chunks(hidden_state[0])
    h1c = h_chunks(hidden_state[1])

    x0_chans = (tuple([0, 1] + [-1] * 30),)
    hc_chans = lambda base: (tuple(range(base, base + 32)),
                             tuple(range(base + 32, base + 64)))

    def dcgru(x_chunks, x_chans, h_chunks_l, Wg, bg, Wc, bc):
        nx = len(x_chunks)
        chans = list(x_chans) + list(hc_chans(
            2 if nx == 1 else 64))
        wg = _adjust_weights(Wg, chans)
        wc = _adjust_weights(Wc, chans)

        base_tbls = list(x_chunks) + list(h_chunks_l)
        y1_0, y1_1 = _diffuse(base_tbls, pk)
        y2_0, y2_1 = _diffuse_pair(y1_0, y1_1, pk)

        def mtabs(ci):
            return [base_tbls[ci], y1_0[ci], y2_0[ci], y1_1[ci], y2_1[ci]]

        nt = 5 * len(base_tbls)
        gate = _make_gate(nt)
        ts = [_flat(t) for ci in range(len(base_tbls)) for t in mtabs(ci)]
        wlist = [wg[ci][m] for ci in range(len(base_tbls)) for m in range(5)]
        rh0, rh1, u = gate(*ts, *wlist, bg.reshape(1, 128).astype(f32),
                           _flat(h_chunks_l[0]), _flat(h_chunks_l[1]))

        rh_tbls = [rh0.reshape(N, 128), rh1.reshape(N, 128)]
        r1_0, r1_1 = _diffuse(rh_tbls, pk)
        r2_0, r2_1 = _diffuse_pair(r1_0, r1_1, pk)

        def rtabs(ci):
            return [rh_tbls[ci], r1_0[ci], r2_0[ci], r1_1[ci], r2_1[ci]]

        ntc = 5 * (nx + 2)
        cand = _make_cand(ntc)
        tsc = [_flat(t) for ci in range(nx) for t in mtabs(ci)]
        tsc += [_flat(t) for ci in range(2) for t in rtabs(ci)]
        wcl = [wc[ci][m] for ci in range(nx) for m in range(5)]
        wcl += [wc[nx + ci][m] for ci in range(2) for m in range(5)]
        hp0, hp1 = cand(*tsc, *wcl, bc.reshape(1, 64).astype(f32), u,
                        _flat(h_chunks_l[0]), _flat(h_chunks_l[1]))
        return hp0.reshape(N, 128), hp1.reshape(N, 128)

    hp0_a, hp0_b = dcgru([x_tbl], x0_chans, h0c, Wg0, bg0, Wc0, bc0)
    hp1_a, hp1_b = dcgru([hp0_a, hp0_b],
                         (tuple(range(0, 32)), tuple(range(32, 64))),
                         h1c, Wg1, bg1, Wc1, bc1)

    def assemble(ca, cb):
        t = jnp.stack([ca.reshape(N, B, 32), cb.reshape(N, B, 32)], axis=2)
        return jnp.transpose(t, (1, 0, 2, 3)).reshape(B, N, 64)

    h0_out = assemble(hp0_a, hp0_b)
    h1_out = assemble(hp1_a, hp1_b)
    return (h1_out, jnp.stack([h0_out, h1_out]))


# speedup vs baseline: 1.6811x; 1.6811x over previous
import functools

import jax
import jax.numpy as jnp
from jax import lax
from jax.experimental import pallas as pl
from jax.experimental.pallas import tpu as pltpu
from jax.experimental.pallas import tpu_sc as plsc

N = 10000
E = 160000
B = 4
NUM_UNITS = 64

_NC = 2
_NS = 16
_EPT = E // _NS
_EK = 200
_NB = _EPT // _EK
_NP = 10240
_RPT = _NP // _NS
_ZR = 64
_FC = 128


@functools.cache
def _make_spmm(ntbl, plan):
    njob = len(plan)
    mesh = plsc.VectorSubcoreMesh(core_axis_name="c", subcore_axis_name="s")

    scratch = [
        pltpu.VMEM_SHARED((_NP, _FC), jnp.float32),
        pltpu.VMEM((_ZR, _FC), jnp.float32),
        pltpu.VMEM((_EK, _FC), jnp.float32),
        pltpu.VMEM((_EK,), jnp.int32),
        pltpu.VMEM((_EK,), jnp.int32),
        pltpu.VMEM((_EK + 16,), jnp.float32),
        pltpu.SemaphoreType.DMA,
    ]

    @functools.partial(
        pl.kernel,
        out_type=tuple(jax.ShapeDtypeStruct((_NP, _FC), jnp.float32)
                       for _ in range(njob)),
        mesh=mesh,
        scratch_types=scratch,
    )
    def spmm(*refs):
        tbls = refs[:ntbl]
        src_h, dst_h, w1_h, w2_h = refs[ntbl:ntbl + 4]
        outs = refs[ntbl + 4:ntbl + 4 + njob]
        acc, zbuf, gbuf, gi, si, wv, semG = refs[ntbl + 4 + njob:]
        cid = lax.axis_index("c")
        sid = lax.axis_index("s")
        zv = jnp.zeros((16,), jnp.float32)

        def zrow(i, _):
            for c in range(_FC // 16):
                zbuf[i, pl.ds(c * 16, 16)] = zv
            return 0

        lax.fori_loop(0, _ZR, zrow, 0)

        for j in range(njob):
            ti, sup = plan[j]
            tbl, out = tbls[ti], outs[j]
            gidx_h = src_h if sup == 0 else dst_h
            sidx_h = dst_h if sup == 0 else src_h
            w_h = w1_h if sup == 0 else w2_h

            @pl.when(cid == (j % _NC))
            def _(tbl=tbl, out=out, gidx_h=gidx_h, sidx_h=sidx_h, w_h=w_h):
                row0 = sid * _RPT
                ebase = sid * _EPT

                for z in range(_RPT // _ZR):
                    pltpu.sync_copy(zbuf, acc.at[pl.ds(row0 + z * _ZR, _ZR)])
                plsc.subcore_barrier()

                def blk(b, _):
                    base = ebase + b * _EK
                    pltpu.sync_copy(gidx_h.at[pl.ds(base, _EK)], gi)
                    pltpu.sync_copy(sidx_h.at[pl.ds(base, _EK)], si)
                    pltpu.sync_copy(w_h.at[pl.ds(base, _EK)],
                                    wv.at[pl.ds(0, _EK)])
                    pltpu.async_copy(tbl.at[gi], gbuf, semG).wait()

                    @plsc.parallel_loop(0, _EK, 1, unroll=4)
                    def erow(i):
                        ws = wv[pl.ds(i, 16)][0]
                        for c in range(_FC // 16):
                            sl = pl.ds(c * 16, 16)
                            gbuf[i, sl] = gbuf[i, sl] * ws

                    pltpu.sync_copy(gbuf, acc.at[si], add=True)
                    return 0

                lax.fori_loop(0, _NB, blk, 0)
                plsc.subcore_barrier()
                pltpu.sync_copy(acc.at[pl.ds(row0, _RPT)],
                                out.at[pl.ds(row0, _RPT)])
                plsc.subcore_barrier()

    return spmm


def _diffuse(tables, pk):
    nt = len(tables)
    plan = tuple((c, s) for c in range(nt) for s in (0, 1))
    fn = _make_spmm(nt, plan)
    outs = fn(*tables, *pk)
    outs = [o[:N] for o in outs]
    ys0 = [outs[2 * c] for c in range(nt)]
    ys1 = [outs[2 * c + 1] for c in range(nt)]
    return ys0, ys1


def _diffuse_pair(tablesA, tablesB, pk):
    nt = len(tablesA)
    assert len(tablesB) == nt
    plan = tuple((c * 2 + s, s) for c in range(nt) for s in (0, 1))
    fn = _make_spmm(2 * nt, plan)
    tbls = []
    for c in range(nt):
        tbls.append(tablesA[c])
        tbls.append(tablesB[c])
    outs = fn(*tbls, *pk)
    outs = [o[:N] for o in outs]
    ysA = [outs[2 * c] for c in range(nt)]
    ysB = [outs[2 * c + 1] for c in range(nt)]
    return ysA, ysB


_M = N * B
_RBLK = 2000


@functools.cache
def _make_gate(nt):
    def body(*refs):
        ts = refs[:nt]
        ws = refs[nt:2 * nt]
        bias = refs[2 * nt]
        h0, h1 = refs[2 * nt + 1], refs[2 * nt + 2]
        rh0, rh1, u = refs[2 * nt + 3:]
        acc = bias[...].astype(jnp.float32) * jnp.ones((_RBLK, 1), jnp.float32)
        for i in range(nt):
            acc += jnp.dot(ts[i][...], ws[i][...],
                           preferred_element_type=jnp.float32)
        g = jax.nn.sigmoid(acc)
        rh0[...] = g[:, :32] * h0[...]
        rh1[...] = g[:, 32:64] * h1[...]
        u[...] = g[:, 64:]

    t_spec = pl.BlockSpec((_RBLK, 32), lambda i: (i, 0))
    w_spec = pl.BlockSpec((32, 128), lambda i: (0, 0))

    return pl.pallas_call(
        body,
        grid=(_M // _RBLK,),
        in_specs=[t_spec] * nt + [w_spec] * nt
        + [pl.BlockSpec((1, 128), lambda i: (0, 0))]
        + [t_spec, t_spec],
        out_specs=[t_spec, t_spec, pl.BlockSpec((_RBLK, 64), lambda i: (i, 0))],
        out_shape=[
            jax.ShapeDtypeStruct((_M, 32), jnp.float32),
            jax.ShapeDtypeStruct((_M, 32), jnp.float32),
            jax.ShapeDtypeStruct((_M, 64), jnp.float32),
        ],
    )


@functools.cache
def _make_cand(nt):
    def body(*refs):
        ts = refs[:nt]
        ws = refs[nt:2 * nt]
        bias = refs[2 * nt]
        u = refs[2 * nt + 1]
        h0, h1 = refs[2 * nt + 2], refs[2 * nt + 3]
        hp0, hp1 = refs[2 * nt + 4:]
        acc = bias[...].astype(jnp.float32) * jnp.ones((_RBLK, 1), jnp.float32)
        for i in range(nt):
            acc += jnp.dot(ts[i][...], ws[i][...],
                           preferred_element_type=jnp.float32)
        c = jnp.tanh(acc)
        uu = u[...]
        hp0[...] = uu[:, :32] * h0[...] + (1.0 - uu[:, :32]) * c[:, :32]
        hp1[...] = uu[:, 32:] * h1[...] + (1.0 - uu[:, 32:]) * c[:, 32:]

    t_spec = pl.BlockSpec((_RBLK, 32), lambda i: (i, 0))
    w_spec = pl.BlockSpec((32, 64), lambda i: (0, 0))

    return pl.pallas_call(
        body,
        grid=(_M // _RBLK,),
        in_specs=[t_spec] * nt + [w_spec] * nt
        + [pl.BlockSpec((1, 64), lambda i: (0, 0))]
        + [pl.BlockSpec((_RBLK, 64), lambda i: (i, 0)), t_spec, t_spec],
        out_specs=[t_spec, t_spec],
        out_shape=[
            jax.ShapeDtypeStruct((_M, 32), jnp.float32),
            jax.ShapeDtypeStruct((_M, 32), jnp.float32),
        ],
    )


def _adjust_weights(W, chans):
    C = W.shape[0] // 5
    out = W.shape[1]
    Wr = W.reshape(C, 5, out)
    Wa = jnp.stack([
        Wr[:, 0] - Wr[:, 2] - Wr[:, 4],
        Wr[:, 1],
        2.0 * Wr[:, 2],
        Wr[:, 3],
        2.0 * Wr[:, 4],
    ])
    res = []
    for idx in chans:
        ia = jnp.array([max(ch, 0) for ch in idx], jnp.int32)
        mask = jnp.array([1.0 if ch >= 0 else 0.0 for ch in idx], W.dtype)
        res.append(jnp.take(Wa, ia, axis=1) * mask[None, :, None])
    return res


def _flat(t):
    return t.reshape(_M, 32)


def kernel(inputs, hidden_state, src, dst, w1, w2, Wg0, bg0, Wc0, bc0, Wg1, bg1, Wc1, bc1):
    f32 = jnp.float32
    pk = (src.astype(jnp.int32), dst.astype(jnp.int32), w1, w2)
    x_nb = jnp.transpose(inputs, (1, 0, 2))
    x_tbl = jnp.pad(x_nb, ((0, 0), (0, 0), (0, 30))).reshape(N, 128)

    def h_chunks(h):
        t = jnp.transpose(h, (1, 0, 2)).reshape(N, B, 2, 32)
        t = jnp.transpose(t, (0, 2, 1, 3))
        return [t[:, 0].reshape(N, 128), t[:, 1].reshape(N, 128)]

    h0c = h_chunks(hidden_state[0])
    h1c = h_chunks(hidden_state[1])

    x0_chans = (tuple([0, 1] + [-1] * 30),)
    hc_chans = lambda base: (tuple(range(base, base + 32)),
                             tuple(range(base + 32, base + 64)))

    def dcgru(x_chunks, x_chans, h_chunks_l, Wg, bg, Wc, bc):
        nx = len(x_chunks)
        chans = list(x_chans) + list(hc_chans(
            2 if nx == 1 else 64))
        wg = _adjust_weights(Wg, chans)
        wc = _adjust_weights(Wc, chans)

        base_tbls = list(x_chunks) + list(h_chunks_l)
        y1_0, y1_1 = _diffuse(base_tbls, pk)
        y2_0, y2_1 = _diffuse_pair(y1_0, y1_1, pk)

        def mtabs(ci):
            return [base_tbls[ci], y1_0[ci], y2_0[ci], y1_1[ci], y2_1[ci]]

        nt = 5 * len(base_tbls)
        gate = _make_gate(nt)
        ts = [_flat(t) for ci in range(len(base_tbls)) for t in mtabs(ci)]
        wlist = [wg[ci][m] for ci in range(len(base_tbls)) for m in range(5)]
        rh0, rh1, u = gate(*ts, *wlist, bg.reshape(1, 128).astype(f32),
                           _flat(h_chunks_l[0]), _flat(h_chunks_l[1]))

        rh_tbls = [rh0.reshape(N, 128), rh1.reshape(N, 128)]
        r1_0, r1_1 = _diffuse(rh_tbls, pk)
        r2_0, r2_1 = _diffuse_pair(r1_0, r1_1, pk)

        def rtabs(ci):
            return [rh_tbls[ci], r1_0[ci], r2_0[ci], r1_1[ci], r2_1[ci]]

        ntc = 5 * (nx + 2)
        cand = _make_cand(ntc)
        tsc = [_flat(t) for ci in range(nx) for t in mtabs(ci)]
        tsc += [_flat(t) for ci in range(2) for t in rtabs(ci)]
        wcl = [wc[ci][m] for ci in range(nx) for m in range(5)]
        wcl += [wc[nx + ci][m] for ci in range(2) for m in range(5)]
        hp0, hp1 = cand(*tsc, *wcl, bc.reshape(1, 64).astype(f32), u,
                        _flat(h_chunks_l[0]), _flat(h_chunks_l[1]))
        return hp0.reshape(N, 128), hp1.reshape(N, 128)

    hp0_a, hp0_b = dcgru([x_tbl], x0_chans, h0c, Wg0, bg0, Wc0, bc0)
    hp1_a, hp1_b = dcgru([hp0_a, hp0_b],
                         (tuple(range(0, 32)), tuple(range(32, 64))),
                         h1c, Wg1, bg1, Wc1, bc1)

    def assemble(ca, cb):
        t = jnp.stack([ca.reshape(N, B, 32), cb.reshape(N, B, 32)], axis=2)
        return jnp.transpose(t, (1, 0, 2, 3)).reshape(B, N, 64)

    h0_out = assemble(hp0_a, hp0_b)
    h1_out = assemble(hp1_a, hp1_b)
    return (h1_out, jnp.stack([h0_out, h1_out]))
